# Initial kernel scaffold; baseline (speedup 1.0000x reference)
import jax
import jax.numpy as jnp
from jax.experimental import pallas as pl


def kernel(x, edge_index, in_degrees, W_self1, W_neigh1, b1, W_self2, W_neigh2, b2):
    raise NotImplementedError("write your pallas kernel here")


# speedup vs baseline: 8.6955x; 8.6955x over previous
import functools

import jax
import jax.numpy as jnp
from jax import lax
from jax.experimental import pallas as pl
from jax.experimental.pallas import tpu as pltpu
from jax.experimental.pallas import tpu_sc as plsc

NC = 2
NS = 16
NW = NC * NS
CH = 125
ZR = 128


def _seg_sum_call(n_pad, d, iters):
    rpt = n_pad // NS
    assert n_pad % NS == 0 and rpt % ZR == 0 and d % 16 == 0

    def body(src_hbm, dst_hbm, tab_hbm, out_hbm, src_v, dst_v, rows_v, zbuf,
             acc, sem):
        c = lax.axis_index("c")
        s = lax.axis_index("s")
        w = s * NC + c

        def zrow(r, _):
            def zcol(k, _):
                zbuf[r, pl.ds(k * 16, 16)] = jnp.zeros((16,), jnp.float32)
                return 0
            lax.fori_loop(0, d // 16, zcol, 0)
            return 0
        lax.fori_loop(0, ZR, zrow, 0)

        row0 = s * rpt

        def zacc(k, _):
            pltpu.sync_copy(zbuf, acc.at[pl.ds(row0 + k * ZR, ZR)])
            return 0
        lax.fori_loop(0, rpt // ZR, zacc, 0)
        plsc.subcore_barrier()

        pltpu.sync_copy(src_hbm.at[pl.ds(w * iters, iters)], src_v)
        pltpu.sync_copy(dst_hbm.at[pl.ds(w * iters, iters)], dst_v)

        def edge(j, _):
            pltpu.async_copy(tab_hbm.at[src_v.at[j]], rows_v, sem).wait()
            pltpu.sync_copy(rows_v, acc.at[dst_v.at[j]], add=True)
            return 0
        lax.fori_loop(0, iters, edge, 0)

        plsc.subcore_barrier()
        pltpu.sync_copy(acc.at[pl.ds(row0, rpt)],
                        out_hbm.at[c, pl.ds(row0, rpt)])

    mesh = plsc.VectorSubcoreMesh(core_axis_name="c", subcore_axis_name="s",
                                  num_cores=NC, num_subcores=NS)
    return pl.kernel(
        body,
        out_type=jax.ShapeDtypeStruct((NC, n_pad, d), jnp.float32),
        mesh=mesh,
        scratch_types=[
            pltpu.VMEM((iters, CH), jnp.int32),
            pltpu.VMEM((iters, CH), jnp.int32),
            pltpu.VMEM((CH, d), jnp.float32),
            pltpu.VMEM((ZR, d), jnp.float32),
            pltpu.VMEM_SHARED((n_pad, d), jnp.float32),
            pltpu.SemaphoreType.DMA,
        ],
        compiler_params=pltpu.CompilerParams(use_tc_tiling_on_sc=False),
    )


def _make_tc1_body(nchunks):
    def body(*refs):
        x_ref = refs[0]
        p_refs = refs[1:1 + nchunks]
        (deg_ref, ws1_ref, wn1_ref, b1_ref, ws2_ref, wn2_ref,
         t2_ref, s2_ref) = refs[1 + nchunks:]
        denom = jnp.maximum(deg_ref[...], 1.0)
        agg = jnp.concatenate([p[0] + p[1] for p in p_refs], axis=-1) / denom
        h = jnp.dot(x_ref[...], ws1_ref[...],
                    preferred_element_type=jnp.float32)
        h = h + jnp.dot(agg, wn1_ref[...], preferred_element_type=jnp.float32)
        h = jnp.maximum(h + b1_ref[...], 0.0)
        t2_ref[...] = jnp.dot(h, wn2_ref[...],
                              preferred_element_type=jnp.float32)
        s2_ref[...] = jnp.dot(h, ws2_ref[...],
                              preferred_element_type=jnp.float32)
    return body


def _tc2_body(s2_ref, p_ref, deg_ref, b2_ref, o_ref):
    denom = jnp.maximum(deg_ref[...], 1.0)
    o_ref[...] = s2_ref[...] + (p_ref[0] + p_ref[1]) / denom + b2_ref[...]


def _pick_bm(n):
    for bm in (512, 400, 256, 200, 128, 80, 40, 16, 8):
        if n % bm == 0:
            return bm
    return n


def kernel(x, edge_index, in_degrees, W_self1, W_neigh1, b1, W_self2,
           W_neigh2, b2):
    n, d_in = x.shape
    d_hid = W_self1.shape[1]
    d_out = W_self2.shape[1]
    e = edge_index.shape[1]

    src = edge_index[0]
    dst = edge_index[1]
    epb = NW * CH * 8
    e_pad = ((e + epb - 1) // epb) * epb
    if e_pad != e:
        src = jnp.concatenate([src, jnp.zeros((e_pad - e,), jnp.int32)])
        dst = jnp.concatenate([dst, jnp.full((e_pad - e,), n, jnp.int32)])
    iters = e_pad // (NW * CH)
    n_pad = ((n + NS * ZR - 1) // (NS * ZR)) * (NS * ZR)
    src2 = src.reshape(NW * iters, CH)
    dst2 = dst.reshape(NW * iters, CH)
    degf = in_degrees.astype(jnp.float32).reshape(n, 1)

    spmem_budget_words = 1_200_000
    max_d = max(16, (spmem_budget_words // n_pad) // 16 * 16)
    nchunks = -(-d_in // max_d)
    d_chunk = -(-(d_in // nchunks) // 16) * 16
    assert d_chunk * nchunks >= d_in and d_in % d_chunk == 0
    nchunks = d_in // d_chunk

    seg1 = _seg_sum_call(n_pad, d_chunk, iters)
    p1s = [seg1(src2, dst2,
                lax.slice_in_dim(x, k * d_chunk, (k + 1) * d_chunk, axis=1))
           for k in range(nchunks)]

    bm = _pick_bm(n)
    grid = (n // bm,)
    t2, s2 = pl.pallas_call(
        _make_tc1_body(nchunks),
        grid=grid,
        in_specs=[
            pl.BlockSpec((bm, d_in), lambda i: (i, 0)),
        ] + [
            pl.BlockSpec((NC, bm, d_chunk), lambda i: (0, i, 0))
            for _ in range(nchunks)
        ] + [
            pl.BlockSpec((bm, 1), lambda i: (i, 0)),
            pl.BlockSpec((d_in, d_hid), lambda i: (0, 0)),
            pl.BlockSpec((d_in, d_hid), lambda i: (0, 0)),
            pl.BlockSpec((1, d_hid), lambda i: (0, 0)),
            pl.BlockSpec((d_hid, d_out), lambda i: (0, 0)),
            pl.BlockSpec((d_hid, d_out), lambda i: (0, 0)),
        ],
        out_specs=[
            pl.BlockSpec((bm, d_out), lambda i: (i, 0)),
            pl.BlockSpec((bm, d_out), lambda i: (i, 0)),
        ],
        out_shape=[jax.ShapeDtypeStruct((n, d_out), jnp.float32)] * 2,
    )(x, *p1s, degf, W_self1, W_neigh1, b1.reshape(1, d_hid), W_self2,
      W_neigh2)

    seg2 = _seg_sum_call(n_pad, d_out, iters)
    p2 = seg2(src2, dst2, t2)

    out = pl.pallas_call(
        _tc2_body,
        grid=grid,
        in_specs=[
            pl.BlockSpec((bm, d_out), lambda i: (i, 0)),
            pl.BlockSpec((NC, bm, d_out), lambda i: (0, i, 0)),
            pl.BlockSpec((bm, 1), lambda i: (i, 0)),
            pl.BlockSpec((1, d_out), lambda i: (0, 0)),
        ],
        out_specs=pl.BlockSpec((bm, d_out), lambda i: (i, 0)),
        out_shape=jax.ShapeDtypeStruct((n, d_out), jnp.float32),
    )(s2, p2, degf, b2.reshape(1, d_out))
    return out


# speedup vs baseline: 12.4932x; 1.4367x over previous
import functools

import jax
import jax.numpy as jnp
from jax import lax
from jax.experimental import pallas as pl
from jax.experimental.pallas import tpu as pltpu
from jax.experimental.pallas import tpu_sc as plsc

NC = 2
NS = 16
NW = NC * NS
CH = 125
ZR = 128


def _seg_sum_call(n_pad, d, iters):
    rpt = n_pad // NS
    assert n_pad % NS == 0 and rpt % ZR == 0 and d % 16 == 0

    assert iters % 2 == 0

    def body(src_hbm, dst_hbm, tab_hbm, out_hbm, src_v, dst_v, rows0, rows1,
             zbuf, acc, sem0, sem1):
        c = lax.axis_index("c")
        s = lax.axis_index("s")
        w = s * NC + c

        def zrow(r, _):
            def zcol(k, _):
                zbuf[r, pl.ds(k * 16, 16)] = jnp.zeros((16,), jnp.float32)
                return 0
            lax.fori_loop(0, d // 16, zcol, 0)
            return 0
        lax.fori_loop(0, ZR, zrow, 0)

        row0 = s * rpt

        def zacc(k, _):
            pltpu.sync_copy(zbuf, acc.at[pl.ds(row0 + k * ZR, ZR)])
            return 0
        lax.fori_loop(0, rpt // ZR, zacc, 0)
        plsc.subcore_barrier()

        pltpu.sync_copy(src_hbm.at[pl.ds(w * iters, iters)], src_v)
        pltpu.sync_copy(dst_hbm.at[pl.ds(w * iters, iters)], dst_v)

        rows = (rows0, rows1)
        sems = (sem0, sem1)

        def wait_gather(k):
            pltpu.make_async_copy(tab_hbm.at[pl.ds(0, CH)], rows[k],
                                  sems[k]).wait()

        pltpu.async_copy(tab_hbm.at[src_v.at[0]], rows0, sem0)

        def edge(i, _):
            j = i * 2
            pltpu.async_copy(tab_hbm.at[src_v.at[j + 1]], rows1, sem1)
            wait_gather(0)
            pltpu.sync_copy(rows0, acc.at[dst_v.at[j]], add=True)

            @pl.when(i < iters // 2 - 1)
            def _():
                pltpu.async_copy(tab_hbm.at[src_v.at[j + 2]], rows0, sem0)
            wait_gather(1)
            pltpu.sync_copy(rows1, acc.at[dst_v.at[j + 1]], add=True)
            return 0
        lax.fori_loop(0, iters // 2, edge, 0)

        plsc.subcore_barrier()
        pltpu.sync_copy(acc.at[pl.ds(row0, rpt)],
                        out_hbm.at[c, pl.ds(row0, rpt)])

    mesh = plsc.VectorSubcoreMesh(core_axis_name="c", subcore_axis_name="s",
                                  num_cores=NC, num_subcores=NS)
    return pl.kernel(
        body,
        out_type=jax.ShapeDtypeStruct((NC, n_pad, d), jnp.float32),
        mesh=mesh,
        scratch_types=[
            pltpu.VMEM((iters, CH), jnp.int32),
            pltpu.VMEM((iters, CH), jnp.int32),
            pltpu.VMEM((CH, d), jnp.float32),
            pltpu.VMEM((CH, d), jnp.float32),
            pltpu.VMEM((ZR, d), jnp.float32),
            pltpu.VMEM_SHARED((n_pad, d), jnp.float32),
            pltpu.SemaphoreType.DMA,
            pltpu.SemaphoreType.DMA,
        ],
        compiler_params=pltpu.CompilerParams(use_tc_tiling_on_sc=False),
    )


def _make_tc1_body(nchunks):
    def body(*refs):
        x_ref = refs[0]
        p_refs = refs[1:1 + nchunks]
        (deg_ref, ws1_ref, wn1_ref, b1_ref, ws2_ref, wn2_ref,
         t2_ref, s2_ref) = refs[1 + nchunks:]
        denom = jnp.maximum(deg_ref[...], 1.0)
        agg = jnp.concatenate([p[0] + p[1] for p in p_refs], axis=-1) / denom
        h = jnp.dot(x_ref[...], ws1_ref[...],
                    preferred_element_type=jnp.float32)
        h = h + jnp.dot(agg, wn1_ref[...], preferred_element_type=jnp.float32)
        h = jnp.maximum(h + b1_ref[...], 0.0)
        t2_ref[...] = jnp.dot(h, wn2_ref[...],
                              preferred_element_type=jnp.float32)
        s2_ref[...] = jnp.dot(h, ws2_ref[...],
                              preferred_element_type=jnp.float32)
    return body


def _tc2_body(s2_ref, p_ref, deg_ref, b2_ref, o_ref):
    denom = jnp.maximum(deg_ref[...], 1.0)
    o_ref[...] = s2_ref[...] + (p_ref[0] + p_ref[1]) / denom + b2_ref[...]


def _pick_bm(n):
    for bm in (512, 400, 256, 200, 128, 80, 40, 16, 8):
        if n % bm == 0:
            return bm
    return n


def kernel(x, edge_index, in_degrees, W_self1, W_neigh1, b1, W_self2,
           W_neigh2, b2):
    n, d_in = x.shape
    d_hid = W_self1.shape[1]
    d_out = W_self2.shape[1]
    e = edge_index.shape[1]

    src = edge_index[0]
    dst = edge_index[1]
    epb = NW * CH * 8
    e_pad = ((e + epb - 1) // epb) * epb
    if e_pad != e:
        src = jnp.concatenate([src, jnp.zeros((e_pad - e,), jnp.int32)])
        dst = jnp.concatenate([dst, jnp.full((e_pad - e,), n, jnp.int32)])
    iters = e_pad // (NW * CH)
    n_pad = ((n + NS * ZR - 1) // (NS * ZR)) * (NS * ZR)
    src2 = src.reshape(NW * iters, CH)
    dst2 = dst.reshape(NW * iters, CH)
    degf = in_degrees.astype(jnp.float32).reshape(n, 1)

    spmem_budget_words = 1_200_000
    max_d = max(16, (spmem_budget_words // n_pad) // 16 * 16)
    nchunks = -(-d_in // max_d)
    d_chunk = -(-(d_in // nchunks) // 16) * 16
    assert d_chunk * nchunks >= d_in and d_in % d_chunk == 0
    nchunks = d_in // d_chunk

    seg1 = _seg_sum_call(n_pad, d_chunk, iters)
    p1s = [seg1(src2, dst2,
                lax.slice_in_dim(x, k * d_chunk, (k + 1) * d_chunk, axis=1))
           for k in range(nchunks)]

    bm = _pick_bm(n)
    grid = (n // bm,)
    t2, s2 = pl.pallas_call(
        _make_tc1_body(nchunks),
        grid=grid,
        in_specs=[
            pl.BlockSpec((bm, d_in), lambda i: (i, 0)),
        ] + [
            pl.BlockSpec((NC, bm, d_chunk), lambda i: (0, i, 0))
            for _ in range(nchunks)
        ] + [
            pl.BlockSpec((bm, 1), lambda i: (i, 0)),
            pl.BlockSpec((d_in, d_hid), lambda i: (0, 0)),
            pl.BlockSpec((d_in, d_hid), lambda i: (0, 0)),
            pl.BlockSpec((1, d_hid), lambda i: (0, 0)),
            pl.BlockSpec((d_hid, d_out), lambda i: (0, 0)),
            pl.BlockSpec((d_hid, d_out), lambda i: (0, 0)),
        ],
        out_specs=[
            pl.BlockSpec((bm, d_out), lambda i: (i, 0)),
            pl.BlockSpec((bm, d_out), lambda i: (i, 0)),
        ],
        out_shape=[jax.ShapeDtypeStruct((n, d_out), jnp.float32)] * 2,
    )(x, *p1s, degf, W_self1, W_neigh1, b1.reshape(1, d_hid), W_self2,
      W_neigh2)

    seg2 = _seg_sum_call(n_pad, d_out, iters)
    p2 = seg2(src2, dst2, t2)

    out = pl.pallas_call(
        _tc2_body,
        grid=grid,
        in_specs=[
            pl.BlockSpec((bm, d_out), lambda i: (i, 0)),
            pl.BlockSpec((NC, bm, d_out), lambda i: (0, i, 0)),
            pl.BlockSpec((bm, 1), lambda i: (i, 0)),
            pl.BlockSpec((1, d_out), lambda i: (0, 0)),
        ],
        out_specs=pl.BlockSpec((bm, d_out), lambda i: (i, 0)),
        out_shape=jax.ShapeDtypeStruct((n, d_out), jnp.float32),
    )(s2, p2, degf, b2.reshape(1, d_out))
    return out


# speedup vs baseline: 12.5092x; 1.0013x over previous
import functools

import jax
import jax.numpy as jnp
from jax import lax
from jax.experimental import pallas as pl
from jax.experimental.pallas import tpu as pltpu
from jax.experimental.pallas import tpu_sc as plsc

NC = 2
NS = 16
NW = NC * NS
CH = 125
ZR = 128


def _seg_sum_call(n_pad, d, iters, nchunks):
    rpt = n_pad // NS
    assert n_pad % NS == 0 and rpt % ZR == 0 and d % 16 == 0
    assert iters % 2 == 0

    def body(*refs):
        src_hbm, dst_hbm = refs[0], refs[1]
        tabs = refs[2:2 + nchunks]
        outs = refs[2 + nchunks:2 + 2 * nchunks]
        (src_v, dst_v, rows0, rows1, zbuf, acc, sem0, sem1) = \
            refs[2 + 2 * nchunks:]
        c = lax.axis_index("c")
        s = lax.axis_index("s")
        w = s * NC + c
        row0 = s * rpt

        def zrow(r, _):
            def zcol(k, _):
                zbuf[r, pl.ds(k * 16, 16)] = jnp.zeros((16,), jnp.float32)
                return 0
            lax.fori_loop(0, d // 16, zcol, 0)
            return 0
        lax.fori_loop(0, ZR, zrow, 0)

        pltpu.sync_copy(src_hbm.at[pl.ds(w * iters, iters)], src_v)
        pltpu.sync_copy(dst_hbm.at[pl.ds(w * iters, iters)], dst_v)

        rows = (rows0, rows1)
        sems = (sem0, sem1)

        for tab_hbm, out_hbm in zip(tabs, outs):
            def zacc(k, _):
                pltpu.sync_copy(zbuf, acc.at[pl.ds(row0 + k * ZR, ZR)])
                return 0
            lax.fori_loop(0, rpt // ZR, zacc, 0)
            plsc.subcore_barrier()

            def wait_gather(k):
                pltpu.make_async_copy(tab_hbm.at[pl.ds(0, CH)], rows[k],
                                      sems[k]).wait()

            pltpu.async_copy(tab_hbm.at[src_v.at[0]], rows0, sem0)

            def edge(i, _):
                j = i * 2
                pltpu.async_copy(tab_hbm.at[src_v.at[j + 1]], rows1, sem1)
                wait_gather(0)
                pltpu.sync_copy(rows0, acc.at[dst_v.at[j]], add=True)

                @pl.when(i < iters // 2 - 1)
                def _():
                    pltpu.async_copy(tab_hbm.at[src_v.at[j + 2]], rows0,
                                     sem0)
                wait_gather(1)
                pltpu.sync_copy(rows1, acc.at[dst_v.at[j + 1]], add=True)
                return 0
            lax.fori_loop(0, iters // 2, edge, 0)

            plsc.subcore_barrier()
            pltpu.sync_copy(acc.at[pl.ds(row0, rpt)],
                            out_hbm.at[c, pl.ds(row0, rpt)])

    mesh = plsc.VectorSubcoreMesh(core_axis_name="c", subcore_axis_name="s",
                                  num_cores=NC, num_subcores=NS)
    return pl.kernel(
        body,
        out_type=[jax.ShapeDtypeStruct((NC, n_pad, d), jnp.float32)
                  for _ in range(nchunks)],
        mesh=mesh,
        scratch_types=[
            pltpu.VMEM((iters, CH), jnp.int32),
            pltpu.VMEM((iters, CH), jnp.int32),
            pltpu.VMEM((CH, d), jnp.float32),
            pltpu.VMEM((CH, d), jnp.float32),
            pltpu.VMEM((ZR, d), jnp.float32),
            pltpu.VMEM_SHARED((n_pad, d), jnp.float32),
            pltpu.SemaphoreType.DMA,
            pltpu.SemaphoreType.DMA,
        ],
        compiler_params=pltpu.CompilerParams(use_tc_tiling_on_sc=False),
    )


def _make_tc1_body(nchunks):
    def body(*refs):
        x_ref = refs[0]
        p_refs = refs[1:1 + nchunks]
        (deg_ref, ws1_ref, wn1_ref, b1_ref, ws2_ref, wn2_ref,
         t2_ref, s2_ref) = refs[1 + nchunks:]
        denom = jnp.maximum(deg_ref[...], 1.0)
        agg = jnp.concatenate([p[0] + p[1] for p in p_refs], axis=-1) / denom
        h = jnp.dot(x_ref[...], ws1_ref[...],
                    preferred_element_type=jnp.float32)
        h = h + jnp.dot(agg, wn1_ref[...], preferred_element_type=jnp.float32)
        h = jnp.maximum(h + b1_ref[...], 0.0)
        t2_ref[...] = jnp.dot(h, wn2_ref[...],
                              preferred_element_type=jnp.float32)
        s2_ref[...] = jnp.dot(h, ws2_ref[...],
                              preferred_element_type=jnp.float32)
    return body


def _tc2_body(s2_ref, p_ref, deg_ref, b2_ref, o_ref):
    denom = jnp.maximum(deg_ref[...], 1.0)
    o_ref[...] = s2_ref[...] + (p_ref[0] + p_ref[1]) / denom + b2_ref[...]


def _pick_bm(n):
    for bm in (512, 400, 256, 200, 128, 80, 40, 16, 8):
        if n % bm == 0:
            return bm
    return n


def kernel(x, edge_index, in_degrees, W_self1, W_neigh1, b1, W_self2,
           W_neigh2, b2):
    n, d_in = x.shape
    d_hid = W_self1.shape[1]
    d_out = W_self2.shape[1]
    e = edge_index.shape[1]

    src = edge_index[0]
    dst = edge_index[1]
    epb = NW * CH * 8
    e_pad = ((e + epb - 1) // epb) * epb
    if e_pad != e:
        src = jnp.concatenate([src, jnp.zeros((e_pad - e,), jnp.int32)])
        dst = jnp.concatenate([dst, jnp.full((e_pad - e,), n, jnp.int32)])
    iters = e_pad // (NW * CH)
    n_pad = ((n + NS * ZR - 1) // (NS * ZR)) * (NS * ZR)
    src2 = src.reshape(NW * iters, CH)
    dst2 = dst.reshape(NW * iters, CH)
    degf = in_degrees.astype(jnp.float32).reshape(n, 1)

    spmem_budget_words = 1_200_000
    max_d = max(16, (spmem_budget_words // n_pad) // 16 * 16)
    nchunks = -(-d_in // max_d)
    d_chunk = -(-(d_in // nchunks) // 16) * 16
    assert d_chunk * nchunks >= d_in and d_in % d_chunk == 0
    nchunks = d_in // d_chunk

    seg1 = _seg_sum_call(n_pad, d_chunk, iters, nchunks)
    xs = [lax.slice_in_dim(x, k * d_chunk, (k + 1) * d_chunk, axis=1)
          for k in range(nchunks)]
    p1s = seg1(src2, dst2, *xs)
    if not isinstance(p1s, (list, tuple)):
        p1s = [p1s]

    bm = _pick_bm(n)
    grid = (n // bm,)
    t2, s2 = pl.pallas_call(
        _make_tc1_body(nchunks),
        grid=grid,
        in_specs=[
            pl.BlockSpec((bm, d_in), lambda i: (i, 0)),
        ] + [
            pl.BlockSpec((NC, bm, d_chunk), lambda i: (0, i, 0))
            for _ in range(nchunks)
        ] + [
            pl.BlockSpec((bm, 1), lambda i: (i, 0)),
            pl.BlockSpec((d_in, d_hid), lambda i: (0, 0)),
            pl.BlockSpec((d_in, d_hid), lambda i: (0, 0)),
            pl.BlockSpec((1, d_hid), lambda i: (0, 0)),
            pl.BlockSpec((d_hid, d_out), lambda i: (0, 0)),
            pl.BlockSpec((d_hid, d_out), lambda i: (0, 0)),
        ],
        out_specs=[
            pl.BlockSpec((bm, d_out), lambda i: (i, 0)),
            pl.BlockSpec((bm, d_out), lambda i: (i, 0)),
        ],
        out_shape=[jax.ShapeDtypeStruct((n, d_out), jnp.float32)] * 2,
    )(x, *p1s, degf, W_self1, W_neigh1, b1.reshape(1, d_hid), W_self2,
      W_neigh2)

    seg2 = _seg_sum_call(n_pad, d_out, iters, 1)
    p2 = seg2(src2, dst2, t2)
    if isinstance(p2, (list, tuple)):
        p2 = p2[0]

    out = pl.pallas_call(
        _tc2_body,
        grid=grid,
        in_specs=[
            pl.BlockSpec((bm, d_out), lambda i: (i, 0)),
            pl.BlockSpec((NC, bm, d_out), lambda i: (0, i, 0)),
            pl.BlockSpec((bm, 1), lambda i: (i, 0)),
            pl.BlockSpec((1, d_out), lambda i: (0, 0)),
        ],
        out_specs=pl.BlockSpec((bm, d_out), lambda i: (i, 0)),
        out_shape=jax.ShapeDtypeStruct((n, d_out), jnp.float32),
    )(s2, p2, degf, b2.reshape(1, d_out))
    return out
